# baseline (speedup 1.0000x reference)
import jax
import jax.numpy as jnp
from jax.experimental import pallas as pl
from jax.experimental.pallas import tpu as pltpu

_B, _C, _H, _W = 16, 192, 64, 64
_HW = _H * _W


def _gain_scale_body(n_ref, gm_ref, x_ref, o_ref):
    b = pl.program_id(0)
    nb = n_ref[b]
    nf = jnp.floor(nb)
    l = nb - nf
    ni = nf.astype(jnp.int32)
    g1 = jnp.abs(gm_ref[pl.ds(ni, 1), :])
    g2 = jnp.abs(gm_ref[pl.ds(ni + 1, 1), :])
    gain = g1 ** (1.0 - l) * g2 ** l
    o_ref[0] = x_ref[0] * gain.reshape(_C, 1, 1)


def kernel(x, n, gain_matrix):
    return pl.pallas_call(
        _gain_scale_body,
        grid=(_B,),
        in_specs=[
            pl.BlockSpec(memory_space=pltpu.SMEM),
            pl.BlockSpec((6, _C), lambda b: (0, 0)),
            pl.BlockSpec((1, _C, _H, _W), lambda b: (b, 0, 0, 0)),
        ],
        out_specs=pl.BlockSpec((1, _C, _H, _W), lambda b: (b, 0, 0, 0)),
        out_shape=jax.ShapeDtypeStruct((_B, _C, _H, _W), jnp.float32),
    )(n, gain_matrix, x)


# speedup vs baseline: 1.5253x; 1.5253x over previous
import jax
import jax.numpy as jnp
from jax.experimental import pallas as pl
from jax.experimental.pallas import tpu as pltpu

_B, _C, _H, _W = 16, 192, 64, 64
_HW = _H * _W


_HR, _WR = 32, 128


def _gain_scale_body(n_ref, gm_ref, x_ref, o_ref):
    b = pl.program_id(0)
    nb = n_ref[b]
    nf = jnp.floor(nb)
    l = nb - nf
    ni = nf.astype(jnp.int32)
    g1 = jnp.abs(gm_ref[pl.ds(ni, 1), :])
    g2 = jnp.abs(gm_ref[pl.ds(ni + 1, 1), :])
    gain = g1 ** (1.0 - l) * g2 ** l
    o_ref[0] = x_ref[0] * gain.reshape(_C, 1, 1)


def kernel(x, n, gain_matrix):
    xr = x.reshape(_B, _C, _HR, _WR)
    out = pl.pallas_call(
        _gain_scale_body,
        grid=(_B,),
        in_specs=[
            pl.BlockSpec(memory_space=pltpu.SMEM),
            pl.BlockSpec((6, _C), lambda b: (0, 0)),
            pl.BlockSpec((1, _C, _HR, _WR), lambda b: (b, 0, 0, 0)),
        ],
        out_specs=pl.BlockSpec((1, _C, _HR, _WR), lambda b: (b, 0, 0, 0)),
        out_shape=jax.ShapeDtypeStruct((_B, _C, _HR, _WR), jnp.float32),
    )(n, gain_matrix, xr)
    return out.reshape(_B, _C, _H, _W)


# speedup vs baseline: 3.6727x; 2.4079x over previous
import jax
import jax.numpy as jnp
from jax.experimental import pallas as pl
from jax.experimental.pallas import tpu as pltpu

_B, _C, _H, _W = 16, 192, 64, 64
_HB = 16


def _gain_scale_body(n_ref, gm_ref, x_ref, o_ref, gain_ref):
    b = pl.program_id(0)
    h = pl.program_id(1)

    @pl.when(h == 0)
    def _compute_gain():
        nb = n_ref[b]
        nf = jnp.floor(nb)
        l = nb - nf
        ni = nf.astype(jnp.int32)
        g1 = jnp.abs(gm_ref[pl.ds(ni, 1), :])
        g2 = jnp.abs(gm_ref[pl.ds(ni + 1, 1), :])
        gain_ref[...] = g1 ** (1.0 - l) * g2 ** l

    o_ref[...] = x_ref[...] * gain_ref[...]


def kernel(x, n, gain_matrix):
    xt = jnp.transpose(x, (0, 2, 3, 1))
    out = pl.pallas_call(
        _gain_scale_body,
        grid=(_B, _H // _HB),
        in_specs=[
            pl.BlockSpec(memory_space=pltpu.SMEM),
            pl.BlockSpec((6, _C), lambda b, h: (0, 0)),
            pl.BlockSpec((1, _HB, _W, _C), lambda b, h: (b, h, 0, 0)),
        ],
        out_specs=pl.BlockSpec((1, _HB, _W, _C), lambda b, h: (b, h, 0, 0)),
        out_shape=jax.ShapeDtypeStruct((_B, _H, _W, _C), jnp.float32),
        scratch_shapes=[pltpu.VMEM((1, _C), jnp.float32)],
    )(n, gain_matrix, xt)
    return jnp.transpose(out, (0, 3, 1, 2))


# speedup vs baseline: 5.6560x; 1.5400x over previous
import jax
import jax.numpy as jnp
from jax.experimental import pallas as pl
from jax.experimental.pallas import tpu as pltpu

_B, _C, _H, _W = 16, 192, 64, 64
_HB = 64


def _gain_scale_body(n_ref, gm_ref, x_ref, o_ref, gain_ref):
    b = pl.program_id(0)
    h = pl.program_id(1)

    @pl.when(h == 0)
    def _compute_gain():
        nb = n_ref[b]
        nf = jnp.floor(nb)
        l = nb - nf
        ni = nf.astype(jnp.int32)
        g1 = jnp.abs(gm_ref[pl.ds(ni, 1), :])
        g2 = jnp.abs(gm_ref[pl.ds(ni + 1, 1), :])
        gain_ref[...] = g1 ** (1.0 - l) * g2 ** l

    o_ref[...] = x_ref[...] * gain_ref[...]


def kernel(x, n, gain_matrix):
    xt = jnp.transpose(x, (0, 2, 3, 1))
    out = pl.pallas_call(
        _gain_scale_body,
        grid=(_B, _H // _HB),
        in_specs=[
            pl.BlockSpec(memory_space=pltpu.SMEM),
            pl.BlockSpec((6, _C), lambda b, h: (0, 0)),
            pl.BlockSpec((1, _HB, _W, _C), lambda b, h: (b, h, 0, 0)),
        ],
        out_specs=pl.BlockSpec((1, _HB, _W, _C), lambda b, h: (b, h, 0, 0)),
        out_shape=jax.ShapeDtypeStruct((_B, _H, _W, _C), jnp.float32),
        scratch_shapes=[pltpu.VMEM((1, _C), jnp.float32)],
    )(n, gain_matrix, xt)
    return jnp.transpose(out, (0, 3, 1, 2))


# speedup vs baseline: 5.8626x; 1.0365x over previous
import jax
import jax.numpy as jnp
from jax.experimental import pallas as pl
from jax.experimental.pallas import tpu as pltpu

_B, _C, _H, _W = 16, 192, 64, 64
_BB = 2


def _gain_scale_body(n_ref, gm_ref, x_ref, o_ref):
    i = pl.program_id(0)
    gains = []
    for j in range(_BB):
        nb = n_ref[_BB * i + j]
        nf = jnp.floor(nb)
        l = nb - nf
        ni = nf.astype(jnp.int32)
        g1 = jnp.abs(gm_ref[pl.ds(ni, 1), :])
        g2 = jnp.abs(gm_ref[pl.ds(ni + 1, 1), :])
        gains.append(g1 ** (1.0 - l) * g2 ** l)
    gain = jnp.concatenate(gains, axis=0)
    o_ref[...] = x_ref[...] * gain.reshape(_BB, 1, 1, _C)


def kernel(x, n, gain_matrix):
    xt = jnp.transpose(x, (0, 2, 3, 1))
    out = pl.pallas_call(
        _gain_scale_body,
        grid=(_B // _BB,),
        in_specs=[
            pl.BlockSpec(memory_space=pltpu.SMEM),
            pl.BlockSpec((6, _C), lambda i: (0, 0)),
            pl.BlockSpec((_BB, _H, _W, _C), lambda i: (i, 0, 0, 0)),
        ],
        out_specs=pl.BlockSpec((_BB, _H, _W, _C), lambda i: (i, 0, 0, 0)),
        out_shape=jax.ShapeDtypeStruct((_B, _H, _W, _C), jnp.float32),
    )(n, gain_matrix, xt)
    return jnp.transpose(out, (0, 3, 1, 2))
